# baseline (speedup 1.0000x reference)
import functools

import jax
import jax.numpy as jnp
from jax import lax
from jax.experimental import pallas as pl
from jax.experimental.pallas import tpu as pltpu
from jax.experimental.pallas import tpu_sc as plsc

_N = 1024
_T = 32
_TNA = 2
_F = 16
_C1 = _N - 1
_C2 = _T - _TNA - 1
_CC = 33
_STEPS = _C1 // _CC
_BLK1 = _CC * _T * _N


def _d1_body(x_ref, o_ref, rank_ref, jp_ref):
    i = pl.program_id(0)

    @pl.when(i == 0)
    def _():
        xb = x_ref[0]
        nan3 = jnp.where(xb != xb, 1.0, 0.0)
        invalid = jnp.max(nan3, axis=-1)
        valid2 = 1.0 - invalid
        ic = jax.lax.broadcasted_iota(jnp.int32, (_N, _N), 0)
        ik = jax.lax.broadcasted_iota(jnp.int32, (_N, _N), 1)
        tri_hi = jnp.where(ik <= ic + 1, 1.0, 0.0)
        tri_lo = jnp.where(ik <= ic, 1.0, 0.0)
        dn = (((1,), (1,)), ((), ()))
        inc_hi = jax.lax.dot_general(
            tri_hi, valid2, dn, preferred_element_type=jnp.float32)
        inc_lo = jax.lax.dot_general(
            tri_lo, valid2, dn, preferred_element_type=jnp.float32)
        rank_ref[...] = jnp.where(inc_hi - inc_lo > 0.5, inc_hi - 1.0, -1.0)
        ju = jax.lax.broadcasted_iota(jnp.int32, (8, 128), 0)
        jl = jax.lax.broadcasted_iota(jnp.int32, (8, 128), 1)
        jp_ref[...] = (ju * 128 + jl).astype(jnp.float32)

    sl = rank_ref[pl.ds(i * _CC, _CC), :]
    slf = sl.reshape(_CC * _T, 1, 1)
    jp = jp_ref[...]
    eq = jp[None, :, :] == slf
    o_ref[...] = jnp.where(eq, 1.0, 0.0).reshape(_BLK1)


def _d2_sc_body(o_hbm, zv, ov):
    wid = lax.axis_index("s") * 2 + lax.axis_index("c")
    for k in range(_N // 16):
        zv[pl.ds(k * 16, 16)] = jnp.zeros((16,), jnp.float32)
        ov[pl.ds(k * 16, 16)] = jnp.ones((16,), jnp.float32)
    for c in range(_C2):
        off = c * (_T * _N) + wid * _N

        @pl.when(wid == c + _TNA + 1)
        def _():
            pltpu.sync_copy(ov, o_hbm.at[pl.ds(off, _N)])

        @pl.when(wid != c + _TNA + 1)
        def _():
            pltpu.sync_copy(zv, o_hbm.at[pl.ds(off, _N)])


def kernel(x):
    x0 = x
    (d1f,) = pl.pallas_call(
        _d1_body,
        grid=(_STEPS,),
        in_specs=[pl.BlockSpec((1, _T, _N, _F), lambda i: (0, 0, 0, 0))],
        out_specs=[pl.BlockSpec((_BLK1,), lambda i: (i,))],
        out_shape=[jax.ShapeDtypeStruct((_C1 * _T * _N,), jnp.float32)],
        scratch_shapes=[
            pltpu.VMEM((_N, _T), jnp.float32),
            pltpu.VMEM((8, 128), jnp.float32),
        ],
        compiler_params=pltpu.CompilerParams(
            dimension_semantics=("arbitrary",)),
    )(x0)
    d2_sc = functools.partial(
        pl.kernel,
        mesh=plsc.VectorSubcoreMesh(core_axis_name="c", subcore_axis_name="s"),
        out_type=jax.ShapeDtypeStruct((_C2 * _T * _N,), jnp.float32),
        scratch_types=[
            pltpu.VMEM((_N,), jnp.float32),
            pltpu.VMEM((_N,), jnp.float32),
        ],
    )(_d2_sc_body)
    d2f = d2_sc()
    d1 = jnp.swapaxes(d1f.reshape(1, _C1, _T * _N), 1, 2)
    d2 = jnp.swapaxes(d2f.reshape(1, _C2, _T * _N), 1, 2)
    return (d1, d2)


# speedup vs baseline: 1.6879x; 1.6879x over previous
import jax
import jax.numpy as jnp
from jax.experimental import pallas as pl
from jax.experimental.pallas import tpu as pltpu

_N = 1024
_T = 32
_TNA = 2
_F = 16
_C1 = _N - 1
_C2 = _T - _TNA - 1
_CC = 33
_STEPS = _C1 // _CC
_BLK1 = _CC * _T * _N


def _d1_body(x_ref, o_ref, rank_ref, jp_ref):
    i = pl.program_id(0)

    @pl.when(i == 0)
    def _():
        xb = x_ref[0]
        nan3 = jnp.where(xb != xb, 1.0, 0.0)
        invalid = jnp.max(nan3, axis=1)
        valid2 = 1.0 - invalid
        ic = jax.lax.broadcasted_iota(jnp.int32, (_N, _N), 0)
        ik = jax.lax.broadcasted_iota(jnp.int32, (_N, _N), 1)
        tri_hi = jnp.where(ik <= ic + 1, 1.0, 0.0)
        tri_lo = jnp.where(ik <= ic, 1.0, 0.0)
        dn = (((1,), (1,)), ((), ()))
        inc_hi = jax.lax.dot_general(
            tri_hi, valid2, dn, preferred_element_type=jnp.float32)
        inc_lo = jax.lax.dot_general(
            tri_lo, valid2, dn, preferred_element_type=jnp.float32)
        rank_ref[...] = jnp.where(inc_hi - inc_lo > 0.5, inc_hi - 1.0, -1.0)
        ju = jax.lax.broadcasted_iota(jnp.int32, (8, 128), 0)
        jl = jax.lax.broadcasted_iota(jnp.int32, (8, 128), 1)
        jp_ref[...] = (ju * 128 + jl).astype(jnp.float32)

    sl = rank_ref[pl.ds(i * _CC, _CC), :]
    slf = sl.reshape(_CC * _T, 1, 1)
    jp = jp_ref[...]
    eq = jp[None, :, :] == slf
    o_ref[...] = jnp.where(eq, 1.0, 0.0).reshape(_BLK1)


def _d2_body(o_ref):
    a = jax.lax.broadcasted_iota(jnp.int32, (_C2 * _T, 8, 128), 0)
    ones = jnp.where(a % _T == a // _T + _TNA + 1, 1.0, 0.0)
    o_ref[...] = ones.reshape(_C2 * _T * _N)


def kernel(x):
    x0 = jnp.swapaxes(x, 2, 3)
    (d1f,) = pl.pallas_call(
        _d1_body,
        grid=(_STEPS,),
        in_specs=[pl.BlockSpec((1, _T, _F, _N), lambda i: (0, 0, 0, 0))],
        out_specs=[pl.BlockSpec((_BLK1,), lambda i: (i,))],
        out_shape=[jax.ShapeDtypeStruct((_C1 * _T * _N,), jnp.float32)],
        scratch_shapes=[
            pltpu.VMEM((_N, _T), jnp.float32),
            pltpu.VMEM((8, 128), jnp.float32),
        ],
        compiler_params=pltpu.CompilerParams(
            dimension_semantics=("arbitrary",)),
    )(x0)
    (d2f,) = pl.pallas_call(
        _d2_body,
        grid=(1,),
        out_specs=[pl.BlockSpec((_C2 * _T * _N,), lambda i: (i,))],
        out_shape=[jax.ShapeDtypeStruct((_C2 * _T * _N,), jnp.float32)],
    )()
    d1 = jnp.swapaxes(d1f.reshape(1, _C1, _T * _N), 1, 2)
    d2 = jnp.swapaxes(d2f.reshape(1, _C2, _T * _N), 1, 2)
    return (d1, d2)
